# baseline (speedup 1.0000x reference)
import functools

import jax
import jax.numpy as jnp
from jax import lax
from jax.experimental import pallas as pl
from jax.experimental.pallas import tpu as pltpu
from jax.experimental.pallas import tpu_sc as plsc

D_MODEL = 4096
N_EXP = 64
K = 8
BLK = 1024

NC = 2
NS = 16
L = 16
NW = NC * NS


def _gate_body(x_ref, w_ref, acc_ref, gsfull_ref, gsc_ref):
    logits = lax.dot_general(
        x_ref[...], w_ref[...],
        dimension_numbers=(((1,), (1,)), ((), ())),
        preferred_element_type=jnp.float32,
    )
    m = jnp.max(logits, axis=1, keepdims=True)
    e = jnp.exp(logits - m)
    gs = e / jnp.sum(e, axis=1, keepdims=True)
    gsfull_ref[...] = gs
    gsc_ref[...] = gs


def _gate_scores_tc(x, W, gs_acc, base, n):
    B = x.shape[0]
    base_blk = base // BLK
    return pl.pallas_call(
        _gate_body,
        grid=(n // BLK,),
        in_specs=[
            pl.BlockSpec((BLK, D_MODEL), lambda i: (base_blk + i, 0)),
            pl.BlockSpec((N_EXP, D_MODEL), lambda i: (0, 0)),
            pl.BlockSpec(memory_space=pl.ANY),
        ],
        out_specs=[
            pl.BlockSpec((BLK, N_EXP), lambda i: (base_blk + i, 0)),
            pl.BlockSpec((BLK, N_EXP), lambda i: (i, 0)),
        ],
        out_shape=[
            jax.ShapeDtypeStruct((B, N_EXP), jnp.float32),
            jax.ShapeDtypeStruct((n, N_EXP), jnp.float32),
        ],
        input_output_aliases={2: 0},
        compiler_params=pltpu.CompilerParams(
            dimension_semantics=("arbitrary",),
        ),
    )(x, W, gs_acc)


def _topk_sc_body(rows_per_w, gs_hbm, ti_hbm, tw_hbm, sc_v, ti_v, tw_v):
    wid = lax.axis_index("s") * NC + lax.axis_index("c")
    base = wid * rows_per_w
    pltpu.sync_copy(gs_hbm.at[pl.ds(base, rows_per_w), :], sc_v)
    lane = lax.broadcasted_iota(jnp.int32, (L,), 0)
    neg_inf_bits = jnp.full((L,), -(2**31), jnp.int32)
    mask_hi = jnp.full((L,), ~63, jnp.int32)

    @plsc.parallel_loop(0, rows_per_w // L, 1, unroll=2)
    def _(g):
        rows = g * L + lane
        best = [neg_inf_bits] * K
        for e in range(N_EXP):
            v = plsc.load_gather(sc_v, [rows, jnp.full((L,), e, jnp.int32)])
            p = (plsc.bitcast(v, jnp.int32) & mask_hi) | jnp.full((L,), 63 - e, jnp.int32)
            for j in range(K):
                nb = jnp.maximum(best[j], p)
                p = jnp.minimum(best[j], p)
                best[j] = nb
        ws = []
        for j in range(K):
            idx = 63 - (best[j] & jnp.full((L,), 63, jnp.int32))
            plsc.store_scatter(ti_v, [rows, jnp.full((L,), j, jnp.int32)], idx)
            ws.append(plsc.load_gather(sc_v, [rows, idx]))
        tot = ws[0]
        for j in range(1, K):
            tot = tot + ws[j]
        denom = tot + jnp.full((L,), 1e-8, jnp.float32)
        for j in range(K):
            plsc.store_scatter(tw_v, [rows, jnp.full((L,), j, jnp.int32)], ws[j] / denom)
    pltpu.sync_copy(ti_v, ti_hbm.at[pl.ds(base, rows_per_w), :])
    pltpu.sync_copy(tw_v, tw_hbm.at[pl.ds(base, rows_per_w), :])


def _topk_sc(gs):
    B = gs.shape[0]
    rows_per_w = B // NW
    return pl.kernel(
        functools.partial(_topk_sc_body, rows_per_w),
        out_type=[
            jax.ShapeDtypeStruct((B, K), jnp.int32),
            jax.ShapeDtypeStruct((B, K), jnp.float32),
        ],
        mesh=plsc.VectorSubcoreMesh(
            core_axis_name="c", subcore_axis_name="s",
            num_cores=NC, num_subcores=NS,
        ),
        scratch_types=[
            pltpu.VMEM((rows_per_w, N_EXP), jnp.float32),
            pltpu.VMEM((rows_per_w, K), jnp.int32),
            pltpu.VMEM((rows_per_w, K), jnp.float32),
        ],
        compiler_params=pltpu.CompilerParams(needs_layout_passes=False),
    )(gs)


CHUNKS = (8192, 8192, 8192, 8192)


@jax.jit
def kernel(x, W):
    B = x.shape[0]
    gs = jnp.zeros((B, N_EXP), jnp.float32)
    ti_parts, tw_parts = [], []
    base = 0
    for n in CHUNKS:
        gs, gs_c = _gate_scores_tc(x, W, gs, base, n)
        ti_c, tw_c = _topk_sc(gs_c)
        ti_parts.append(ti_c)
        tw_parts.append(tw_c)
        base += n
    ti = jnp.concatenate(ti_parts, axis=0)
    tw = jnp.concatenate(tw_parts, axis=0)
    return gs, ti, tw


# speedup vs baseline: 1.2034x; 1.2034x over previous
import functools

import jax
import jax.numpy as jnp
from jax import lax
from jax.experimental import pallas as pl
from jax.experimental.pallas import tpu as pltpu
from jax.experimental.pallas import tpu_sc as plsc

D_MODEL = 4096
N_EXP = 64
K = 8
BLK = 1024

NC = 2
NS = 16
L = 16
NW = NC * NS


def _gate_body(x_ref, w_ref, gs_ref):
    logits = lax.dot_general(
        x_ref[...], w_ref[...],
        dimension_numbers=(((1,), (1,)), ((), ())),
        preferred_element_type=jnp.float32,
    )
    m = jnp.max(logits, axis=1, keepdims=True)
    e = jnp.exp(logits - m)
    gs_ref[...] = e / jnp.sum(e, axis=1, keepdims=True)


def _gate_scores_tc(x, W, base, n):
    base_blk = base // BLK
    return pl.pallas_call(
        _gate_body,
        grid=(n // BLK,),
        in_specs=[
            pl.BlockSpec((BLK, D_MODEL), lambda i: (base_blk + i, 0)),
            pl.BlockSpec((N_EXP, D_MODEL), lambda i: (0, 0)),
        ],
        out_specs=pl.BlockSpec((BLK, N_EXP), lambda i: (i, 0)),
        out_shape=jax.ShapeDtypeStruct((n, N_EXP), jnp.float32),
        compiler_params=pltpu.CompilerParams(
            dimension_semantics=("arbitrary",),
        ),
    )(x, W)


def _topk_sc_body(rows_per_w, gs_hbm, ti_hbm, tw_hbm, sc_v, ti_v, tw_v):
    wid = lax.axis_index("s") * NC + lax.axis_index("c")
    base = wid * rows_per_w
    pltpu.sync_copy(gs_hbm.at[pl.ds(base, rows_per_w), :], sc_v)
    lane = lax.broadcasted_iota(jnp.int32, (L,), 0)
    neg_inf_bits = jnp.full((L,), -(2**31), jnp.int32)
    mask_hi = jnp.full((L,), ~63, jnp.int32)

    @plsc.parallel_loop(0, rows_per_w // L, 1, unroll=2)
    def _(g):
        rows = g * L + lane
        best = [neg_inf_bits] * K
        for e in range(N_EXP):
            v = plsc.load_gather(sc_v, [rows, jnp.full((L,), e, jnp.int32)])
            p = (plsc.bitcast(v, jnp.int32) & mask_hi) | jnp.full((L,), 63 - e, jnp.int32)
            for j in range(K):
                nb = jnp.maximum(best[j], p)
                p = jnp.minimum(best[j], p)
                best[j] = nb
        ws = []
        for j in range(K):
            idx = 63 - (best[j] & jnp.full((L,), 63, jnp.int32))
            plsc.store_scatter(ti_v, [rows, jnp.full((L,), j, jnp.int32)], idx)
            ws.append(plsc.load_gather(sc_v, [rows, idx]))
        tot = ws[0]
        for j in range(1, K):
            tot = tot + ws[j]
        denom = tot + jnp.full((L,), 1e-8, jnp.float32)
        for j in range(K):
            plsc.store_scatter(tw_v, [rows, jnp.full((L,), j, jnp.int32)], ws[j] / denom)
    pltpu.sync_copy(ti_v, ti_hbm.at[pl.ds(base, rows_per_w), :])
    pltpu.sync_copy(tw_v, tw_hbm.at[pl.ds(base, rows_per_w), :])


def _topk_sc(gs):
    B = gs.shape[0]
    rows_per_w = B // NW
    return pl.kernel(
        functools.partial(_topk_sc_body, rows_per_w),
        out_type=[
            jax.ShapeDtypeStruct((B, K), jnp.int32),
            jax.ShapeDtypeStruct((B, K), jnp.float32),
        ],
        mesh=plsc.VectorSubcoreMesh(
            core_axis_name="c", subcore_axis_name="s",
            num_cores=NC, num_subcores=NS,
        ),
        scratch_types=[
            pltpu.VMEM((rows_per_w, N_EXP), jnp.float32),
            pltpu.VMEM((rows_per_w, K), jnp.int32),
            pltpu.VMEM((rows_per_w, K), jnp.float32),
        ],
        compiler_params=pltpu.CompilerParams(needs_layout_passes=False),
    )(gs)


CHUNKS = (8192, 8192, 8192, 8192)


@jax.jit
def kernel(x, W):
    gs_parts, ti_parts, tw_parts = [], [], []
    base = 0
    for n in CHUNKS:
        gs_c = _gate_scores_tc(x, W, base, n)
        ti_c, tw_c = _topk_sc(gs_c)
        gs_parts.append(gs_c)
        ti_parts.append(ti_c)
        tw_parts.append(tw_c)
        base += n
    gs = jnp.concatenate(gs_parts, axis=0)
    ti = jnp.concatenate(ti_parts, axis=0)
    tw = jnp.concatenate(tw_parts, axis=0)
    return gs, ti, tw
